# baseline (speedup 1.0000x reference)
import functools
import math

import jax
import jax.numpy as jnp
from jax import lax
from jax.experimental import pallas as pl
from jax.experimental.pallas import tpu as pltpu
from jax.experimental.pallas import tpu_sc as plsc

N = 10000
E = 320000
D = 128

NC = 2
NS = 16
L = 16
NW = NC * NS
C = 64
EPW = E // NW
CH = 2 * math.ceil(EPW / (2 * C))
EPW_PAD = CH * C
E_PAD = EPW_PAD * NW
N_PAD = 10240
RPT = N_PAD // NS
DEG_W = 16

_MESH = plsc.VectorSubcoreMesh(core_axis_name="c", subcore_axis_name="s")


@functools.partial(
    pl.kernel,
    out_type=jax.ShapeDtypeStruct((NC, N_PAD, DEG_W), jnp.float32),
    mesh=_MESH,
    scratch_types=[
        pltpu.VMEM((CH, C), jnp.int32),
        pltpu.VMEM((C, DEG_W), jnp.float32),
        pltpu.VMEM_SHARED((N_PAD, DEG_W), jnp.float32),
    ],
)
def _deg_kernel(coli_hbm, zeros_hbm, out_hbm, col_v, ones_v, dacc_sh):
    cid = lax.axis_index("c")
    sid = lax.axis_index("s")
    wid = sid * NC + cid

    e0 = jnp.where(lax.iota(jnp.int32, L) == 0,
                   jnp.full((L,), 1.0, jnp.float32),
                   jnp.full((L,), 0.0, jnp.float32))

    def _fill(i, carry):
        ones_v[i] = e0
        return carry
    lax.fori_loop(0, C, _fill, 0)

    pltpu.sync_copy(zeros_hbm.at[pl.ds(sid * RPT, RPT)],
                    dacc_sh.at[pl.ds(sid * RPT, RPT)])
    pltpu.sync_copy(coli_hbm.at[wid], col_v)
    plsc.subcore_barrier()

    def _body(j, carry):
        pltpu.sync_copy(ones_v, dacc_sh.at[col_v.at[j]], add=True)
        return carry
    lax.fori_loop(0, CH, _body, 0)

    plsc.subcore_barrier()
    pltpu.sync_copy(dacc_sh.at[pl.ds(sid * RPT, RPT)],
                    out_hbm.at[cid, pl.ds(sid * RPT, RPT)])


@functools.partial(
    pl.kernel,
    out_type=jax.ShapeDtypeStruct((NC, N_PAD, D), jnp.float32),
    mesh=_MESH,
    scratch_types=[
        pltpu.VMEM((CH // 2 + 1, 2 * C), jnp.int32),
        pltpu.VMEM((CH, C), jnp.int32),
        pltpu.VMEM((C, D), jnp.float32),
        pltpu.VMEM((C, D), jnp.float32),
        pltpu.VMEM_SHARED((N_PAD, D), jnp.float32),
        pltpu.SemaphoreType.DMA,
        pltpu.SemaphoreType.DMA,
    ],
)
def _agg_kernel(g_hbm, rowi_hbm, coli_hbm, zeros_hbm, out_hbm,
                row_v, col_v, rows0_v, rows1_v, acc_sh, gsem0, gsem1):
    cid = lax.axis_index("c")
    sid = lax.axis_index("s")
    wid = sid * NC + cid

    pltpu.sync_copy(zeros_hbm.at[pl.ds(sid * RPT, RPT)],
                    acc_sh.at[pl.ds(sid * RPT, RPT)])
    pltpu.sync_copy(rowi_hbm.at[wid], row_v)
    pltpu.sync_copy(coli_hbm.at[wid], col_v)
    plsc.subcore_barrier()

    def _ridx(j_half, off):
        return row_v.at[j_half, pl.ds(off, C)]

    pltpu.async_copy(g_hbm.at[_ridx(0, 0)], rows0_v, gsem0)

    def _body(jj, carry):
        j0 = 2 * jj
        j1 = j0 + 1
        pltpu.async_copy(g_hbm.at[_ridx(jj, C)], rows1_v, gsem1)
        pltpu.make_async_copy(g_hbm.at[_ridx(jj, 0)], rows0_v, gsem0).wait()
        pltpu.sync_copy(rows0_v, acc_sh.at[col_v.at[j0]], add=True)
        pltpu.async_copy(g_hbm.at[_ridx(jj + 1, 0)], rows0_v, gsem0)
        pltpu.make_async_copy(g_hbm.at[_ridx(jj, C)], rows1_v, gsem1).wait()
        pltpu.sync_copy(rows1_v, acc_sh.at[col_v.at[j1]], add=True)
        return carry
    lax.fori_loop(0, CH // 2, _body, 0)
    pltpu.make_async_copy(g_hbm.at[_ridx(CH // 2, 0)], rows0_v, gsem0).wait()

    plsc.subcore_barrier()
    pltpu.sync_copy(acc_sh.at[pl.ds(sid * RPT, RPT)],
                    out_hbm.at[cid, pl.ds(sid * RPT, RPT)])


BM = 1000


def _mm_body(x_ref, w_ref, d0_ref, d1_ref, o_ref):
    deg = d0_ref[:, 0:1] + d1_ref[:, 0:1] + 1.0
    dinv = lax.rsqrt(deg)
    o_ref[...] = jnp.dot(x_ref[...], w_ref[...],
                         preferred_element_type=jnp.float32) * dinv


def _final_body(a0_ref, a1_ref, g_ref, d0_ref, d1_ref, b_ref, o_ref):
    deg = d0_ref[:, 0:1] + d1_ref[:, 0:1] + 1.0
    dinv = lax.rsqrt(deg)
    s = dinv * (a0_ref[...] + a1_ref[...] + g_ref[...]) + b_ref[...]
    o_ref[...] = jnp.maximum(s, 0.0)


def kernel(x, edge_index, W, b):
    ei = edge_index.astype(jnp.int32)
    row = ei[0]
    col = ei[1]
    pad = E_PAD - E
    dummy_cols = N + (jnp.arange(pad, dtype=jnp.int32) % (N_PAD - N))
    rowp = jnp.concatenate([row, jnp.zeros((pad,), jnp.int32)])
    rowp = rowp.reshape(NW, CH // 2, 2 * C)
    rowp = jnp.concatenate([rowp, jnp.zeros((NW, 1, 2 * C), jnp.int32)], axis=1)
    colp = jnp.concatenate([col, dummy_cols]).reshape(NW, CH, C)

    zeros_deg = jnp.zeros((N_PAD, DEG_W), jnp.float32)
    zeros_acc = jnp.zeros((N_PAD, D), jnp.float32)

    degp = _deg_kernel(colp, zeros_deg)

    g = pl.pallas_call(
        _mm_body,
        grid=(N // BM,),
        in_specs=[
            pl.BlockSpec((BM, D), lambda i: (i, 0)),
            pl.BlockSpec((D, D), lambda i: (0, 0)),
            pl.BlockSpec((BM, DEG_W), lambda i: (i, 0)),
            pl.BlockSpec((BM, DEG_W), lambda i: (i, 0)),
        ],
        out_specs=pl.BlockSpec((BM, D), lambda i: (i, 0)),
        out_shape=jax.ShapeDtypeStruct((N, D), jnp.float32),
    )(x, W, degp[0], degp[1])

    acc = _agg_kernel(g, rowp, colp, zeros_acc)

    out = pl.pallas_call(
        _final_body,
        grid=(N // BM,),
        in_specs=[
            pl.BlockSpec((BM, D), lambda i: (i, 0)),
            pl.BlockSpec((BM, D), lambda i: (i, 0)),
            pl.BlockSpec((BM, D), lambda i: (i, 0)),
            pl.BlockSpec((BM, DEG_W), lambda i: (i, 0)),
            pl.BlockSpec((BM, DEG_W), lambda i: (i, 0)),
            pl.BlockSpec((1, D), lambda i: (0, 0)),
        ],
        out_specs=pl.BlockSpec((BM, D), lambda i: (i, 0)),
        out_shape=jax.ShapeDtypeStruct((N, D), jnp.float32),
    )(acc[0], acc[1], g, degp[0], degp[1], b.reshape(1, D))

    return out


# speedup vs baseline: 1.4540x; 1.4540x over previous
import functools
import math

import jax
import jax.numpy as jnp
from jax import lax
from jax.experimental import pallas as pl
from jax.experimental.pallas import tpu as pltpu
from jax.experimental.pallas import tpu_sc as plsc

N = 10000
E = 320000
D = 128

NC = 2
NS = 16
NW = NC * NS
C = 128
N_PAD = 10240
RPT = N_PAD // NS
DEG_W = 16

CH_TOT = math.ceil(E / (C * NS))
CH0 = math.ceil(CH_TOT * 0.655)
CH1 = CH_TOT - CH0
CHMAX = max(CH0, CH1)
E_PAD = CH_TOT * NS * C

CHD = math.ceil(E / (C * NW))
E_PADD = CHD * NW * C

_MESH = plsc.VectorSubcoreMesh(core_axis_name="c", subcore_axis_name="s")


@functools.partial(
    pl.kernel,
    out_type=jax.ShapeDtypeStruct((NC, N_PAD, DEG_W), jnp.float32),
    mesh=_MESH,
    scratch_types=[
        pltpu.VMEM((CHD, C), jnp.int32),
        pltpu.VMEM((C, DEG_W), jnp.float32),
        pltpu.VMEM_SHARED((N_PAD, DEG_W), jnp.float32),
    ],
)
def _deg_kernel(coli_hbm, zeros_hbm, out_hbm, col_v, ones_v, dacc_sh):
    cid = lax.axis_index("c")
    sid = lax.axis_index("s")
    wid = sid * NC + cid

    e0 = jnp.where(lax.iota(jnp.int32, 16) == 0,
                   jnp.full((16,), 1.0, jnp.float32),
                   jnp.full((16,), 0.0, jnp.float32))

    def _fill(i, carry):
        ones_v[i] = e0
        return carry
    lax.fori_loop(0, C, _fill, 0)

    pltpu.sync_copy(zeros_hbm.at[pl.ds(sid * RPT, RPT)],
                    dacc_sh.at[pl.ds(sid * RPT, RPT)])
    pltpu.sync_copy(coli_hbm.at[wid], col_v)
    plsc.subcore_barrier()

    def _body(j, carry):
        pltpu.sync_copy(ones_v, dacc_sh.at[col_v.at[j]], add=True)
        return carry
    lax.fori_loop(0, CHD, _body, 0)

    plsc.subcore_barrier()
    pltpu.sync_copy(dacc_sh.at[pl.ds(sid * RPT, RPT)],
                    out_hbm.at[cid, pl.ds(sid * RPT, RPT)])


@functools.partial(
    pl.kernel,
    out_type=jax.ShapeDtypeStruct((NC, N_PAD, D), jnp.float32),
    mesh=_MESH,
    scratch_types=[
        pltpu.VMEM((CHMAX, C), jnp.int32),
        pltpu.VMEM((CHMAX, C), jnp.int32),
        pltpu.VMEM((C, D), jnp.float32),
        pltpu.VMEM_SHARED((N_PAD, D), jnp.float32),
        pltpu.SemaphoreType.DMA,
    ],
)
def _agg_kernel(g_hbm, rowi_hbm, coli_hbm, zeros_hbm, out_hbm,
                row_v, col_v, rows_v, acc_sh, gsem):
    cid = lax.axis_index("c")
    sid = lax.axis_index("s")
    wid = sid * NC + cid
    nch = lax.select(cid == 0, jnp.int32(CH0), jnp.int32(CH1))

    pltpu.sync_copy(zeros_hbm.at[pl.ds(sid * RPT, RPT)],
                    acc_sh.at[pl.ds(sid * RPT, RPT)])
    pltpu.sync_copy(rowi_hbm.at[wid], row_v)
    pltpu.sync_copy(coli_hbm.at[wid], col_v)
    plsc.subcore_barrier()

    def _body(j, carry):
        pltpu.async_copy(g_hbm.at[row_v.at[j]], rows_v, gsem).wait()
        pltpu.sync_copy(rows_v, acc_sh.at[col_v.at[j]], add=True)
        return carry
    lax.fori_loop(0, nch, _body, 0)

    plsc.subcore_barrier()
    pltpu.sync_copy(acc_sh.at[pl.ds(sid * RPT, RPT)],
                    out_hbm.at[cid, pl.ds(sid * RPT, RPT)])


BM = 1000


def _mm_body(x_ref, w_ref, d0_ref, d1_ref, o_ref):
    deg = d0_ref[:, 0:1] + d1_ref[:, 0:1] + 1.0
    dinv = lax.rsqrt(deg)
    o_ref[...] = jnp.dot(x_ref[...], w_ref[...],
                         preferred_element_type=jnp.float32) * dinv


def _final_body(a0_ref, a1_ref, g_ref, d0_ref, d1_ref, b_ref, o_ref):
    deg = d0_ref[:, 0:1] + d1_ref[:, 0:1] + 1.0
    dinv = lax.rsqrt(deg)
    s = dinv * (a0_ref[...] + a1_ref[...] + g_ref[...]) + b_ref[...]
    o_ref[...] = jnp.maximum(s, 0.0)


def _split_edges(arr, fill):
    flat = jnp.concatenate([arr, fill])
    a = flat[:NS * CH0 * C].reshape(NS, CH0, C)
    b = flat[NS * CH0 * C:].reshape(NS, CH1, C)
    a = jnp.pad(a, ((0, 0), (0, CHMAX - CH0), (0, 0)))
    b = jnp.pad(b, ((0, 0), (0, CHMAX - CH1), (0, 0)))
    return jnp.stack([a, b], axis=1).reshape(NW, CHMAX, C)


def kernel(x, edge_index, W, b):
    ei = edge_index.astype(jnp.int32)
    row = ei[0]
    col = ei[1]
    npad = E_PAD - E
    dummy_cols = N + (jnp.arange(npad, dtype=jnp.int32) % (N_PAD - N))
    rowp = _split_edges(row, jnp.zeros((npad,), jnp.int32))
    colp = _split_edges(col, dummy_cols)

    npadd = E_PADD - E
    dummy_d = N + (jnp.arange(npadd, dtype=jnp.int32) % (N_PAD - N))
    colpd = jnp.concatenate([col, dummy_d]).reshape(NW, CHD, C)

    zeros_deg = jnp.zeros((N_PAD, DEG_W), jnp.float32)
    zeros_acc = jnp.zeros((N_PAD, D), jnp.float32)

    degp = _deg_kernel(colpd, zeros_deg)

    g = pl.pallas_call(
        _mm_body,
        grid=(N // BM,),
        in_specs=[
            pl.BlockSpec((BM, D), lambda i: (i, 0)),
            pl.BlockSpec((D, D), lambda i: (0, 0)),
            pl.BlockSpec((BM, DEG_W), lambda i: (i, 0)),
            pl.BlockSpec((BM, DEG_W), lambda i: (i, 0)),
        ],
        out_specs=pl.BlockSpec((BM, D), lambda i: (i, 0)),
        out_shape=jax.ShapeDtypeStruct((N, D), jnp.float32),
    )(x, W, degp[0], degp[1])

    acc = _agg_kernel(g, rowp, colp, zeros_acc)

    out = pl.pallas_call(
        _final_body,
        grid=(N // BM,),
        in_specs=[
            pl.BlockSpec((BM, D), lambda i: (i, 0)),
            pl.BlockSpec((BM, D), lambda i: (i, 0)),
            pl.BlockSpec((BM, D), lambda i: (i, 0)),
            pl.BlockSpec((BM, DEG_W), lambda i: (i, 0)),
            pl.BlockSpec((BM, DEG_W), lambda i: (i, 0)),
            pl.BlockSpec((1, D), lambda i: (0, 0)),
        ],
        out_specs=pl.BlockSpec((BM, D), lambda i: (i, 0)),
        out_shape=jax.ShapeDtypeStruct((N, D), jnp.float32),
    )(acc[0], acc[1], g, degp[0], degp[1], b.reshape(1, D))

    return out


# speedup vs baseline: 1.4666x; 1.0087x over previous
import functools
import math

import jax
import jax.numpy as jnp
from jax import lax
from jax.experimental import pallas as pl
from jax.experimental.pallas import tpu as pltpu
from jax.experimental.pallas import tpu_sc as plsc

N = 10000
E = 320000
D = 128

NC = 2
NS = 16
NW = NC * NS
C = 128
N_PAD = 10240
RPT = N_PAD // NS
DEG_W = 16

CH_TOT = math.ceil(E / (C * NS))
CH0 = math.ceil(CH_TOT * 0.637)
CH1 = CH_TOT - CH0
CHMAX = max(CH0, CH1)
E_PAD = CH_TOT * NS * C

CHD = 2 * math.ceil(E / (2 * C * NW))
E_PADD = CHD * NW * C

_MESH = plsc.VectorSubcoreMesh(core_axis_name="c", subcore_axis_name="s")


@functools.partial(
    pl.kernel,
    out_type=jax.ShapeDtypeStruct((NC, N_PAD, DEG_W), jnp.float32),
    mesh=_MESH,
    scratch_types=[
        pltpu.VMEM((CHD + 1, C), jnp.int32),
        pltpu.VMEM((C, DEG_W), jnp.float32),
        pltpu.VMEM_SHARED((N_PAD, DEG_W), jnp.float32),
        pltpu.SemaphoreType.DMA,
        pltpu.SemaphoreType.DMA,
    ],
)
def _deg_kernel(coli_hbm, zeros_hbm, out_hbm, col_v, ones_v, dacc_sh,
                dsem0, dsem1):
    cid = lax.axis_index("c")
    sid = lax.axis_index("s")
    wid = sid * NC + cid

    e0 = jnp.where(lax.iota(jnp.int32, 16) == 0,
                   jnp.full((16,), 1.0, jnp.float32),
                   jnp.full((16,), 0.0, jnp.float32))

    def _fill(i, carry):
        ones_v[i] = e0
        return carry
    lax.fori_loop(0, C, _fill, 0)

    pltpu.sync_copy(zeros_hbm.at[pl.ds(sid * RPT, RPT)],
                    dacc_sh.at[pl.ds(sid * RPT, RPT)])
    pltpu.sync_copy(coli_hbm.at[wid], col_v)
    plsc.subcore_barrier()

    def _body(j, carry):
        pltpu.sync_copy(ones_v, dacc_sh.at[col_v.at[j]], add=True)
        return carry
    lax.fori_loop(0, CHD, _body, 0)

    plsc.subcore_barrier()
    pltpu.sync_copy(dacc_sh.at[pl.ds(sid * RPT, RPT)],
                    out_hbm.at[cid, pl.ds(sid * RPT, RPT)])


@functools.partial(
    pl.kernel,
    out_type=jax.ShapeDtypeStruct((NC, N_PAD, D), jnp.float32),
    mesh=_MESH,
    scratch_types=[
        pltpu.VMEM((CHMAX, C), jnp.int32),
        pltpu.VMEM((CHMAX, C), jnp.int32),
        pltpu.VMEM((C, D), jnp.float32),
        pltpu.VMEM_SHARED((N_PAD, D), jnp.float32),
        pltpu.SemaphoreType.DMA,
    ],
)
def _agg_kernel(g_hbm, rowi_hbm, coli_hbm, zeros_hbm, out_hbm,
                row_v, col_v, rows_v, acc_sh, gsem):
    cid = lax.axis_index("c")
    sid = lax.axis_index("s")
    wid = sid * NC + cid
    nch = lax.select(cid == 0, jnp.int32(CH0), jnp.int32(CH1))

    pltpu.sync_copy(zeros_hbm.at[pl.ds(sid * RPT, RPT)],
                    acc_sh.at[pl.ds(sid * RPT, RPT)])
    pltpu.sync_copy(rowi_hbm.at[wid], row_v)
    pltpu.sync_copy(coli_hbm.at[wid], col_v)
    plsc.subcore_barrier()

    def _body(j, carry):
        pltpu.async_copy(g_hbm.at[row_v.at[j]], rows_v, gsem).wait()
        pltpu.sync_copy(rows_v, acc_sh.at[col_v.at[j]], add=True)
        return carry
    lax.fori_loop(0, nch, _body, 0)

    plsc.subcore_barrier()
    pltpu.sync_copy(acc_sh.at[pl.ds(sid * RPT, RPT)],
                    out_hbm.at[cid, pl.ds(sid * RPT, RPT)])


BM = 1000


def _mm_body(x_ref, w_ref, d0_ref, d1_ref, o_ref):
    deg = d0_ref[:, 0:1] + d1_ref[:, 0:1] + 1.0
    dinv = lax.rsqrt(deg)
    o_ref[...] = jnp.dot(x_ref[...], w_ref[...],
                         preferred_element_type=jnp.float32) * dinv


def _final_body(a0_ref, a1_ref, g_ref, d0_ref, d1_ref, b_ref, o_ref):
    deg = d0_ref[:, 0:1] + d1_ref[:, 0:1] + 1.0
    dinv = lax.rsqrt(deg)
    s = dinv * (a0_ref[...] + a1_ref[...] + g_ref[...]) + b_ref[...]
    o_ref[...] = jnp.maximum(s, 0.0)


def _split_edges(arr, fill):
    flat = jnp.concatenate([arr, fill])
    a = flat[:NS * CH0 * C].reshape(NS, CH0, C)
    b = flat[NS * CH0 * C:].reshape(NS, CH1, C)
    a = jnp.pad(a, ((0, 0), (0, CHMAX - CH0), (0, 0)))
    b = jnp.pad(b, ((0, 0), (0, CHMAX - CH1), (0, 0)))
    return jnp.stack([a, b], axis=1).reshape(NW, CHMAX, C)


def kernel(x, edge_index, W, b):
    ei = edge_index.astype(jnp.int32)
    row = ei[0]
    col = ei[1]
    npad = E_PAD - E
    dummy_cols = N + (jnp.arange(npad, dtype=jnp.int32) % (N_PAD - N))
    rowp = _split_edges(row, jnp.zeros((npad,), jnp.int32))
    colp = _split_edges(col, dummy_cols)

    npadd = E_PADD - E
    dummy_d = N + (jnp.arange(npadd, dtype=jnp.int32) % (N_PAD - N))
    colpd = jnp.concatenate([col, dummy_d]).reshape(NW, CHD, C)
    extra = N + (jnp.arange(NW * C, dtype=jnp.int32) % (N_PAD - N))
    colpd = jnp.concatenate([colpd, extra.reshape(NW, 1, C)], axis=1)

    zeros_deg = jnp.zeros((N_PAD, DEG_W), jnp.float32)
    zeros_acc = jnp.zeros((N_PAD, D), jnp.float32)

    degp = _deg_kernel(colpd, zeros_deg)

    g = pl.pallas_call(
        _mm_body,
        grid=(N // BM,),
        in_specs=[
            pl.BlockSpec((BM, D), lambda i: (i, 0)),
            pl.BlockSpec((D, D), lambda i: (0, 0)),
            pl.BlockSpec((BM, DEG_W), lambda i: (i, 0)),
            pl.BlockSpec((BM, DEG_W), lambda i: (i, 0)),
        ],
        out_specs=pl.BlockSpec((BM, D), lambda i: (i, 0)),
        out_shape=jax.ShapeDtypeStruct((N, D), jnp.float32),
    )(x, W, degp[0], degp[1])

    acc = _agg_kernel(g, rowp, colp, zeros_acc)

    out = pl.pallas_call(
        _final_body,
        grid=(N // BM,),
        in_specs=[
            pl.BlockSpec((BM, D), lambda i: (i, 0)),
            pl.BlockSpec((BM, D), lambda i: (i, 0)),
            pl.BlockSpec((BM, D), lambda i: (i, 0)),
            pl.BlockSpec((BM, DEG_W), lambda i: (i, 0)),
            pl.BlockSpec((BM, DEG_W), lambda i: (i, 0)),
            pl.BlockSpec((1, D), lambda i: (0, 0)),
        ],
        out_specs=pl.BlockSpec((BM, D), lambda i: (i, 0)),
        out_shape=jax.ShapeDtypeStruct((N, D), jnp.float32),
    )(acc[0], acc[1], g, degp[0], degp[1], b.reshape(1, D))

    return out


# speedup vs baseline: 1.4677x; 1.0007x over previous
import functools
import math

import jax
import jax.numpy as jnp
from jax import lax
from jax.experimental import pallas as pl
from jax.experimental.pallas import tpu as pltpu
from jax.experimental.pallas import tpu_sc as plsc

N = 10000
E = 320000
D = 128

NC = 2
NS = 16
NW = NC * NS
C = 128
N_PAD = 10240
RPT = N_PAD // NS
DEG_W = 16

CH_TOT = math.ceil(E / (C * NS))
CH0 = 100
CH1 = CH_TOT - CH0
CHMAX = max(CH0, CH1)
E_PAD = CH_TOT * NS * C

_MESH = plsc.VectorSubcoreMesh(core_axis_name="c", subcore_axis_name="s")


@functools.partial(
    pl.kernel,
    out_type=jax.ShapeDtypeStruct((NC, N_PAD, DEG_W), jnp.float32),
    mesh=_MESH,
    scratch_types=[
        pltpu.VMEM((CHMAX, C), jnp.int32),
        pltpu.VMEM((C, DEG_W), jnp.float32),
        pltpu.VMEM_SHARED((N_PAD, DEG_W), jnp.float32),
    ],
)
def _deg_kernel(coli_hbm, zeros_hbm, out_hbm, col_v, ones_v, dacc_sh):
    cid = lax.axis_index("c")
    sid = lax.axis_index("s")
    wid = sid * NC + cid
    nch = lax.select(cid == 0, jnp.int32(CH0), jnp.int32(CH1))

    e0 = jnp.where(lax.iota(jnp.int32, 16) == 0,
                   jnp.full((16,), 1.0, jnp.float32),
                   jnp.full((16,), 0.0, jnp.float32))

    def _fill(i, carry):
        ones_v[i] = e0
        return carry
    lax.fori_loop(0, C, _fill, 0)

    pltpu.sync_copy(zeros_hbm.at[pl.ds(sid * RPT, RPT)],
                    dacc_sh.at[pl.ds(sid * RPT, RPT)])
    pltpu.sync_copy(coli_hbm.at[wid], col_v)
    plsc.subcore_barrier()

    def _body(j, carry):
        pltpu.sync_copy(ones_v, dacc_sh.at[col_v.at[j]], add=True)
        return carry
    lax.fori_loop(0, nch, _body, 0)

    plsc.subcore_barrier()
    pltpu.sync_copy(dacc_sh.at[pl.ds(sid * RPT, RPT)],
                    out_hbm.at[cid, pl.ds(sid * RPT, RPT)])


@functools.partial(
    pl.kernel,
    out_type=jax.ShapeDtypeStruct((NC, N_PAD, D), jnp.float32),
    mesh=_MESH,
    scratch_types=[
        pltpu.VMEM((CHMAX, C), jnp.int32),
        pltpu.VMEM((CHMAX, C), jnp.int32),
        pltpu.VMEM((C, D), jnp.float32),
        pltpu.VMEM_SHARED((N_PAD, D), jnp.float32),
        pltpu.SemaphoreType.DMA,
    ],
)
def _agg_kernel(g_hbm, rowi_hbm, coli_hbm, zeros_hbm, out_hbm,
                row_v, col_v, rows_v, acc_sh, gsem):
    cid = lax.axis_index("c")
    sid = lax.axis_index("s")
    wid = sid * NC + cid
    nch = lax.select(cid == 0, jnp.int32(CH0), jnp.int32(CH1))

    pltpu.sync_copy(zeros_hbm.at[pl.ds(sid * RPT, RPT)],
                    acc_sh.at[pl.ds(sid * RPT, RPT)])
    pltpu.sync_copy(rowi_hbm.at[wid], row_v)
    pltpu.sync_copy(coli_hbm.at[wid], col_v)
    plsc.subcore_barrier()

    def _body(j, carry):
        pltpu.async_copy(g_hbm.at[row_v.at[j]], rows_v, gsem).wait()
        pltpu.sync_copy(rows_v, acc_sh.at[col_v.at[j]], add=True)
        return carry
    lax.fori_loop(0, nch, _body, 0)

    plsc.subcore_barrier()
    pltpu.sync_copy(acc_sh.at[pl.ds(sid * RPT, RPT)],
                    out_hbm.at[cid, pl.ds(sid * RPT, RPT)])


BM = 1000


def _mm_body(x_ref, w_ref, d0_ref, d1_ref, o_ref):
    deg = d0_ref[0, :, 0:1] + d1_ref[0, :, 0:1] + 1.0
    dinv = lax.rsqrt(deg)
    o_ref[...] = jnp.dot(x_ref[...], w_ref[...],
                         preferred_element_type=jnp.float32) * dinv


def _final_body(a0_ref, a1_ref, g_ref, d0_ref, d1_ref, b_ref, o_ref):
    deg = d0_ref[0, :, 0:1] + d1_ref[0, :, 0:1] + 1.0
    dinv = lax.rsqrt(deg)
    s = dinv * (a0_ref[0] + a1_ref[0] + g_ref[...]) + b_ref[...]
    o_ref[...] = jnp.maximum(s, 0.0)


def _split_edges(arr, fill):
    flat = jnp.concatenate([arr, fill])
    a = flat[:NS * CH0 * C].reshape(NS, CH0, C)
    b = flat[NS * CH0 * C:].reshape(NS, CH1, C)
    a = jnp.pad(a, ((0, 0), (0, CHMAX - CH0), (0, 0)))
    b = jnp.pad(b, ((0, 0), (0, CHMAX - CH1), (0, 0)))
    return jnp.stack([a, b], axis=1).reshape(NW, CHMAX, C)


def kernel(x, edge_index, W, b):
    ei = edge_index.astype(jnp.int32)
    row = ei[0]
    col = ei[1]
    npad = E_PAD - E
    dummy_cols = N + (jnp.arange(npad, dtype=jnp.int32) % (N_PAD - N))
    rowp = _split_edges(row, jnp.zeros((npad,), jnp.int32))
    colp = _split_edges(col, dummy_cols)

    zeros_deg = jnp.zeros((N_PAD, DEG_W), jnp.float32)
    zeros_acc = jnp.zeros((N_PAD, D), jnp.float32)

    degp = _deg_kernel(colp, zeros_deg)

    g = pl.pallas_call(
        _mm_body,
        grid=(N // BM,),
        in_specs=[
            pl.BlockSpec((BM, D), lambda i: (i, 0)),
            pl.BlockSpec((D, D), lambda i: (0, 0)),
            pl.BlockSpec((1, BM, DEG_W), lambda i: (0, i, 0)),
            pl.BlockSpec((1, BM, DEG_W), lambda i: (1, i, 0)),
        ],
        out_specs=pl.BlockSpec((BM, D), lambda i: (i, 0)),
        out_shape=jax.ShapeDtypeStruct((N, D), jnp.float32),
    )(x, W, degp, degp)

    acc = _agg_kernel(g, rowp, colp, zeros_acc)

    out = pl.pallas_call(
        _final_body,
        grid=(N // BM,),
        in_specs=[
            pl.BlockSpec((1, BM, D), lambda i: (0, i, 0)),
            pl.BlockSpec((1, BM, D), lambda i: (1, i, 0)),
            pl.BlockSpec((BM, D), lambda i: (i, 0)),
            pl.BlockSpec((1, BM, DEG_W), lambda i: (0, i, 0)),
            pl.BlockSpec((1, BM, DEG_W), lambda i: (1, i, 0)),
            pl.BlockSpec((1, D), lambda i: (0, 0)),
        ],
        out_specs=pl.BlockSpec((BM, D), lambda i: (i, 0)),
        out_shape=jax.ShapeDtypeStruct((N, D), jnp.float32),
    )(acc, acc, g, degp, degp, b.reshape(1, D))

    return out
